# Initial kernel scaffold; baseline (speedup 1.0000x reference)
import jax
import jax.numpy as jnp
from jax.experimental import pallas as pl


def kernel(x, edge_index, W1, b1, W2, b2, W3, b3):
    raise NotImplementedError("write your pallas kernel here")


# speedup vs baseline: 68.6899x; 68.6899x over previous
import functools

import jax
import jax.numpy as jnp
from jax import lax
from jax.experimental import pallas as pl
from jax.experimental.pallas import tpu as pltpu
from jax.experimental.pallas import tpu_sc as plsc

N = 10000
E = 320000
D = 128

NC = 2
NS = 16
NW = NC * NS

EPW = E // NW
CHK = 128
NCH = 82
EPW_PAD = NCH * CHK
PAIRS = 40
SH_ROWS = 10240
RPS = SH_ROWS // NS
ZB = 64


def _build_sc_agg():
  mesh = plsc.VectorSubcoreMesh(core_axis_name="c", subcore_axis_name="s")

  @functools.partial(
      pl.kernel,
      out_type=jax.ShapeDtypeStruct((NC * SH_ROWS, D), jnp.float32),
      mesh=mesh,
      scratch_types=[
          pltpu.VMEM((EPW_PAD,), jnp.int32),
          pltpu.VMEM((2, CHK), jnp.int32),
          pltpu.VMEM((CHK, D), jnp.float32),
          pltpu.VMEM((CHK, D), jnp.float32),
          pltpu.VMEM_SHARED((SH_ROWS, D), jnp.float32),
          pltpu.SemaphoreType.DMA,
          pltpu.SemaphoreType.DMA,
          pltpu.SemaphoreType.DMA,
          pltpu.SemaphoreType.DMA,
      ],
  )
  def sc_agg(g_hbm, src_hbm, dst_hbm, out_hbm,
             src_v, dstb, rows0, rows1, acc_sh, semg0, semg1, semi0, semi1):
    c = lax.axis_index("c")
    s = lax.axis_index("s")
    wid = c * NS + s

    pltpu.sync_copy(src_hbm.at[wid], src_v)

    zvec = jnp.zeros((16,), jnp.float32)

    def zrow(i, carry):
      for k in range(D // 16):
        rows0[i, pl.ds(k * 16, 16)] = zvec
      return carry

    lax.fori_loop(jnp.int32(0), jnp.int32(ZB), zrow, jnp.int32(0))
    for r in range(RPS // ZB):
      pltpu.sync_copy(rows0.at[pl.ds(0, ZB)],
                      acc_sh.at[pl.ds(s * RPS + r * ZB, ZB)])
    plsc.subcore_barrier()

    def gidx(j):
      return src_v.at[pl.ds(j * CHK, CHK)]

    def didx(j):
      return dst_hbm.at[wid, pl.ds(j * CHK, CHK)]

    i0 = jnp.int32(0)
    i1 = jnp.int32(1)
    pltpu.async_copy(g_hbm.at[gidx(i0)], rows0, semg0)
    pltpu.async_copy(g_hbm.at[gidx(i1)], rows1, semg1)
    pltpu.async_copy(didx(i0), dstb.at[i0], semi0)
    pltpu.async_copy(didx(i1), dstb.at[i1], semi1)

    def half(j, rows, semg, semi, slot):
      pltpu.make_async_copy(g_hbm.at[gidx(j)], rows, semg).wait()
      pltpu.make_async_copy(didx(j), dstb.at[slot], semi).wait()
      pltpu.sync_copy(rows, acc_sh.at[dstb.at[slot]], add=True)
      pltpu.async_copy(didx(j + 2), dstb.at[slot], semi)
      pltpu.async_copy(g_hbm.at[gidx(j + 2)], rows, semg)

    def step(jj, carry):
      j0 = jnp.int32(2) * jj
      half(j0, rows0, semg0, semi0, i0)
      half(j0 + 1, rows1, semg1, semi1, i1)
      return carry

    lax.fori_loop(jnp.int32(0), jnp.int32(PAIRS), step, jnp.int32(0))
    jl0 = jnp.int32(NCH - 2)
    jl1 = jnp.int32(NCH - 1)
    pltpu.make_async_copy(g_hbm.at[gidx(jl0)], rows0, semg0).wait()
    pltpu.make_async_copy(g_hbm.at[gidx(jl1)], rows1, semg1).wait()
    pltpu.make_async_copy(didx(jl0), dstb.at[i0], semi0).wait()
    pltpu.make_async_copy(didx(jl1), dstb.at[i1], semi1).wait()
    plsc.subcore_barrier()

    base = c * SH_ROWS + s * RPS
    for r in range(RPS // 128):
      pltpu.sync_copy(acc_sh.at[pl.ds(s * RPS + r * 128, 128)],
                      out_hbm.at[pl.ds(base + r * 128, 128)])

  return sc_agg


def _build_sc_deg():
  mesh = plsc.VectorSubcoreMesh(core_axis_name="c", subcore_axis_name="s")

  @functools.partial(
      pl.kernel,
      out_type=jax.ShapeDtypeStruct((NC * SH_ROWS, D), jnp.float32),
      mesh=mesh,
      scratch_types=[
          pltpu.VMEM((NCH, CHK), jnp.int32),
          pltpu.VMEM((CHK, D), jnp.float32),
          pltpu.VMEM((ZB, D), jnp.float32),
          pltpu.VMEM_SHARED((SH_ROWS, D), jnp.float32),
      ],
  )
  def sc_deg(dst_hbm, out_hbm, dst_v, ones2, z2, deg_sh):
    c = lax.axis_index("c")
    s = lax.axis_index("s")
    wid = c * NS + s

    pltpu.sync_copy(dst_hbm.at[wid], dst_v)

    zvec = jnp.zeros((16,), jnp.float32)
    ovec = jnp.full((16,), 1.0, jnp.float32)

    def fill(i, carry):
      for k in range(D // 16):
        ones2[i, pl.ds(k * 16, 16)] = ovec
      return carry

    lax.fori_loop(jnp.int32(0), jnp.int32(CHK), fill, jnp.int32(0))

    def zfill(i, carry):
      for k in range(D // 16):
        z2[i, pl.ds(k * 16, 16)] = zvec
      return carry

    lax.fori_loop(jnp.int32(0), jnp.int32(ZB), zfill, jnp.int32(0))
    for r in range(RPS // ZB):
      pltpu.sync_copy(z2, deg_sh.at[pl.ds(s * RPS + r * ZB, ZB)])
    plsc.subcore_barrier()

    def count(j, carry):
      pltpu.sync_copy(ones2, deg_sh.at[dst_v.at[j]], add=True)
      return carry

    lax.fori_loop(jnp.int32(0), jnp.int32(NCH), count, jnp.int32(0))
    plsc.subcore_barrier()

    base = c * SH_ROWS + s * RPS
    for r in range(RPS // 128):
      pltpu.sync_copy(deg_sh.at[pl.ds(s * RPS + r * 128, 128)],
                      out_hbm.at[pl.ds(base + r * 128, 128)])

  return sc_deg


_TC_BR = 2000


def _tc_entry_body(x_ref, w_ref, dinv_ref, o_ref):
  o_ref[...] = jnp.dot(x_ref[...], w_ref[...],
                       preferred_element_type=jnp.float32) * dinv_ref[...]


def _tc_mid_body(agg_ref, g_ref, dinv_ref, b_ref, w_ref, o_ref):
  a = agg_ref[0] + agg_ref[1] + g_ref[...]
  h = jnp.maximum(a * dinv_ref[...] + b_ref[...], 0.0)
  o_ref[...] = jnp.dot(h, w_ref[...],
                       preferred_element_type=jnp.float32) * dinv_ref[...]


def _tc_exit_body(agg_ref, g_ref, dinv_ref, b_ref, o_ref):
  a = agg_ref[0] + agg_ref[1] + g_ref[...]
  o_ref[...] = a * dinv_ref[...] + b_ref[...]


def _tc_specs():
  full_w = pl.BlockSpec((D, D), lambda i: (jnp.int32(0), jnp.int32(0)))
  rows = pl.BlockSpec((_TC_BR, D), lambda i: (i, jnp.int32(0)))
  dinv = pl.BlockSpec((_TC_BR, 1), lambda i: (i, jnp.int32(0)))
  bias = pl.BlockSpec((1, D), lambda i: (jnp.int32(0), jnp.int32(0)))
  agg = pl.BlockSpec((NC, _TC_BR, D),
                     lambda i: (jnp.int32(0), i, jnp.int32(0)))
  return full_w, rows, dinv, bias, agg


def _tc_entry(x, w, dinv_col):
  full_w, rows, dinv, _, _ = _tc_specs()
  return pl.pallas_call(
      _tc_entry_body,
      grid=(N // _TC_BR,),
      in_specs=[rows, full_w, dinv],
      out_specs=rows,
      out_shape=jax.ShapeDtypeStruct((N, D), jnp.float32),
  )(x, w, dinv_col)


def _tc_mid(agg, g, dinv_col, b2d, w):
  full_w, rows, dinv, bias, aggs = _tc_specs()
  return pl.pallas_call(
      _tc_mid_body,
      grid=(N // _TC_BR,),
      in_specs=[aggs, rows, dinv, bias, full_w],
      out_specs=rows,
      out_shape=jax.ShapeDtypeStruct((N, D), jnp.float32),
  )(agg, g, dinv_col, b2d, w)


def _tc_exit(agg, g, dinv_col, b2d):
  _, rows, dinv, bias, aggs = _tc_specs()
  return pl.pallas_call(
      _tc_exit_body,
      grid=(N // _TC_BR,),
      in_specs=[aggs, rows, dinv, bias],
      out_specs=rows,
      out_shape=jax.ShapeDtypeStruct((N, D), jnp.float32),
  )(agg, g, dinv_col, b2d)


def kernel(x, edge_index, W1, b1, W2, b2, W3, b3):
  out_dtype = jnp.result_type(x.dtype, b3.dtype)
  x = x.astype(jnp.float32)
  W1, b1 = W1.astype(jnp.float32), b1.astype(jnp.float32)
  W2, b2 = W2.astype(jnp.float32), b2.astype(jnp.float32)
  W3, b3 = W3.astype(jnp.float32), b3.astype(jnp.float32)
  src = edge_index[0].astype(jnp.int32).reshape(NW, EPW)
  dst = edge_index[1].astype(jnp.int32).reshape(NW, EPW)
  pad = EPW_PAD - EPW
  srcw = jnp.pad(src, ((0, 0), (0, pad)))
  dstw = jnp.pad(dst, ((0, 0), (0, pad)), constant_values=N)
  dstw3 = dstw.reshape(NW, NCH, CHK)
  sc_deg = _build_sc_deg()
  sc_agg = _build_sc_agg()

  degp = sc_deg(dstw3).reshape(NC, SH_ROWS, D)
  deg = degp[0, :N, 0] + degp[1, :N, 0] + 1.0
  dinv_col = lax.rsqrt(deg)[:, None]

  b1_2d = b1.reshape(1, D)
  b2_2d = b2.reshape(1, D)
  b3_2d = b3.reshape(1, D)

  g1 = _tc_entry(x, W1, dinv_col)
  agg1 = sc_agg(g1, srcw, dstw).reshape(NC, SH_ROWS, D)
  g2 = _tc_mid(agg1, g1, dinv_col, b1_2d, W2)
  agg2 = sc_agg(g2, srcw, dstw).reshape(NC, SH_ROWS, D)
  g3 = _tc_mid(agg2, g2, dinv_col, b2_2d, W3)
  agg3 = sc_agg(g3, srcw, dstw).reshape(NC, SH_ROWS, D)
  return _tc_exit(agg3, g3, dinv_col, b3_2d).astype(out_dtype)
